# Initial kernel scaffold; baseline (speedup 1.0000x reference)
import jax
import jax.numpy as jnp
from jax.experimental import pallas as pl


def kernel(x, edge_index, W1, b1, W2, b2, W3, b3):
    raise NotImplementedError("write your pallas kernel here")


# speedup vs baseline: 17.8930x; 17.8930x over previous
import functools

import jax
import jax.numpy as jnp
from jax import lax
from jax.experimental import pallas as pl
from jax.experimental.pallas import tpu as pltpu
from jax.experimental.pallas import tpu_sc as plsc

_NC = 2
_NS = 16
_NW = _NC * _NS
_CH = 128


def _sc_mesh():
    return plsc.VectorSubcoreMesh(
        core_axis_name="c", subcore_axis_name="s",
        num_cores=_NC, num_subcores=_NS)


def _degree_partials(dst, ones_src, zeros_src, n):
    e = dst.shape[0]
    ew = e // _NW
    n_full, tail = divmod(ew, _CH)
    nz, ztail = divmod(n, _CH)
    nzc = (nz + _NS - 1) // _NS

    @functools.partial(
        pl.kernel,
        out_type=jax.ShapeDtypeStruct((_NC * n,), jnp.float32),
        mesh=_sc_mesh(),
        scratch_types=[
            pltpu.VMEM((_CH,), jnp.int32),
            pltpu.VMEM((tail,), jnp.int32) if tail else None,
            pltpu.VMEM((_CH,), jnp.float32),
            pltpu.VMEM((_CH,), jnp.float32),
            pltpu.VMEM_SHARED((n,), jnp.float32),
        ],
        compiler_params=pltpu.CompilerParams(use_tc_tiling_on_sc=False),
    )
    def deg_kernel(dst_hbm, ones_hbm, zeros_hbm, out_hbm,
                   didx, didx_t, ones_v, zeros_v, acc):
        c = lax.axis_index("c")
        s = lax.axis_index("s")
        w = c * _NS + s
        pltpu.sync_copy(ones_hbm, ones_v)
        pltpu.sync_copy(zeros_hbm, zeros_v)

        def zacc(j, carry):
            i = s + j * _NS

            @pl.when(i < nz)
            def _():
                pltpu.sync_copy(zeros_v, acc.at[pl.ds(i * _CH, _CH)])
            return carry
        lax.fori_loop(0, nzc, zacc, 0)
        if ztail:
            @pl.when(s == 0)
            def _():
                pltpu.sync_copy(zeros_v.at[pl.ds(0, ztail)],
                                acc.at[pl.ds(nz * _CH, ztail)])
        plsc.subcore_barrier()

        base_w = w * ew

        def body(i, carry):
            pltpu.sync_copy(dst_hbm.at[pl.ds(base_w + i * _CH, _CH)], didx)
            pltpu.sync_copy(ones_v, acc.at[didx], add=True)
            return carry
        lax.fori_loop(0, n_full, body, 0)
        if tail:
            base = base_w + n_full * _CH
            pltpu.sync_copy(dst_hbm.at[pl.ds(base, tail)], didx_t)
            pltpu.sync_copy(ones_v.at[pl.ds(0, tail)], acc.at[didx_t],
                            add=True)
        plsc.subcore_barrier()

        def out_body(j, carry):
            i = s + j * _NS

            @pl.when(i < nz)
            def _():
                pltpu.sync_copy(acc.at[pl.ds(i * _CH, _CH)], zeros_v)
                pltpu.sync_copy(zeros_v,
                                out_hbm.at[pl.ds(c * n + i * _CH, _CH)])
            return carry
        lax.fori_loop(0, nzc, out_body, 0)
        if ztail:
            @pl.when(s == 0)
            def _():
                pltpu.sync_copy(acc.at[pl.ds(nz * _CH, ztail)],
                                zeros_v.at[pl.ds(0, ztail)])
                pltpu.sync_copy(zeros_v.at[pl.ds(0, ztail)],
                                out_hbm.at[pl.ds(c * n + nz * _CH, ztail)])

    return deg_kernel(dst, ones_src, zeros_src)


def _aggregate(src, dst, z, zero_rows, n):
    e = src.shape[0]
    f = z.shape[1]
    ew = e // _NW
    n_full, tail = divmod(ew, _CH)
    nz, ztail = divmod(n, _CH)
    nzc = (nz + _NS - 1) // _NS

    @functools.partial(
        pl.kernel,
        out_type=jax.ShapeDtypeStruct((_NC * n, f), jnp.float32),
        mesh=_sc_mesh(),
        scratch_types=[
            pltpu.VMEM((_CH,), jnp.int32),
            pltpu.VMEM((_CH,), jnp.int32),
            pltpu.VMEM((tail,), jnp.int32) if tail else None,
            pltpu.VMEM((tail,), jnp.int32) if tail else None,
            pltpu.VMEM((_CH, f), jnp.float32),
            pltpu.VMEM((tail, f), jnp.float32) if tail else None,
            pltpu.VMEM((_CH, f), jnp.float32),
            pltpu.VMEM_SHARED((n, f), jnp.float32),
            pltpu.SemaphoreType.DMA,
        ],
        compiler_params=pltpu.CompilerParams(use_tc_tiling_on_sc=False),
    )
    def agg_kernel(src_hbm, dst_hbm, z_hbm, zrows_hbm, out_hbm,
                   sidx, didx, sidx_t, didx_t, rows, rows_t, zrows, acc, sem):
        c = lax.axis_index("c")
        s = lax.axis_index("s")
        w = c * _NS + s
        pltpu.sync_copy(zrows_hbm, zrows)

        def zacc(j, carry):
            i = s + j * _NS

            @pl.when(i < nz)
            def _():
                pltpu.sync_copy(zrows, acc.at[pl.ds(i * _CH, _CH)])
            return carry
        lax.fori_loop(0, nzc, zacc, 0)
        if ztail:
            @pl.when(s == 0)
            def _():
                pltpu.sync_copy(zrows.at[pl.ds(0, ztail)],
                                acc.at[pl.ds(nz * _CH, ztail)])
        plsc.subcore_barrier()

        base_w = w * ew

        def body(i, carry):
            base = base_w + i * _CH
            pltpu.sync_copy(src_hbm.at[pl.ds(base, _CH)], sidx)
            pltpu.sync_copy(dst_hbm.at[pl.ds(base, _CH)], didx)
            pltpu.async_copy(z_hbm.at[sidx], rows, sem).wait()
            pltpu.sync_copy(rows, acc.at[didx], add=True)
            return carry
        lax.fori_loop(0, n_full, body, 0)
        if tail:
            base = base_w + n_full * _CH
            pltpu.sync_copy(src_hbm.at[pl.ds(base, tail)], sidx_t)
            pltpu.sync_copy(dst_hbm.at[pl.ds(base, tail)], didx_t)
            pltpu.async_copy(z_hbm.at[sidx_t], rows_t, sem).wait()
            pltpu.sync_copy(rows_t, acc.at[didx_t], add=True)
        plsc.subcore_barrier()

        def out_body(j, carry):
            i = s + j * _NS

            @pl.when(i < nz)
            def _():
                pltpu.sync_copy(acc.at[pl.ds(i * _CH, _CH)], rows)
                pltpu.sync_copy(rows,
                                out_hbm.at[pl.ds(c * n + i * _CH, _CH)])
            return carry
        lax.fori_loop(0, nzc, out_body, 0)
        if ztail:
            @pl.when(s == 0)
            def _():
                pltpu.sync_copy(acc.at[pl.ds(nz * _CH, ztail)],
                                rows.at[pl.ds(0, ztail)])
                pltpu.sync_copy(rows.at[pl.ds(0, ztail)],
                                out_hbm.at[pl.ds(c * n + nz * _CH, ztail)])

    return agg_kernel(src, dst, z, zero_rows)


def _dinv(d_ref):
    return lax.rsqrt(d_ref[:, 0:1] + d_ref[:, 1:2] + 1.0)


_R = 1000


def _tc_first(x, w1, deg_t):
    n, d_in = x.shape
    f = w1.shape[1]

    def body(x_ref, w_ref, d_ref, o_ref):
        z = jnp.dot(x_ref[...], w_ref[...],
                    preferred_element_type=jnp.float32)
        o_ref[...] = z * _dinv(d_ref)

    return pl.pallas_call(
        body,
        grid=(n // _R,),
        in_specs=[
            pl.BlockSpec((_R, d_in), lambda i: (i, 0)),
            pl.BlockSpec((d_in, f), lambda i: (0, 0)),
            pl.BlockSpec((_R, _NC), lambda i: (i, 0)),
        ],
        out_specs=pl.BlockSpec((_R, f), lambda i: (i, 0)),
        out_shape=jax.ShapeDtypeStruct((n, f), jnp.float32),
    )(x, w1, deg_t)


def _tc_mid(p0, p1, zp, deg_t, b, w_next):
    n, f = zp.shape
    f2 = w_next.shape[1]

    def body(p0_ref, p1_ref, z_ref, d_ref, b_ref, w_ref, o_ref):
        dinv = _dinv(d_ref)
        h = (p0_ref[...] + p1_ref[...] + z_ref[...]) * dinv + b_ref[...]
        h = jnp.maximum(h, 0.0)
        o_ref[...] = jnp.dot(h, w_ref[...],
                             preferred_element_type=jnp.float32) * dinv

    return pl.pallas_call(
        body,
        grid=(n // _R,),
        in_specs=[
            pl.BlockSpec((_R, f), lambda i: (i, 0)),
            pl.BlockSpec((_R, f), lambda i: (i, 0)),
            pl.BlockSpec((_R, f), lambda i: (i, 0)),
            pl.BlockSpec((_R, _NC), lambda i: (i, 0)),
            pl.BlockSpec((1, f), lambda i: (0, 0)),
            pl.BlockSpec((f, f2), lambda i: (0, 0)),
        ],
        out_specs=pl.BlockSpec((_R, f2), lambda i: (i, 0)),
        out_shape=jax.ShapeDtypeStruct((n, f2), jnp.float32),
    )(p0, p1, zp, deg_t, b, w_next)


def _tc_prelast(p0, p1, zp, deg_t, b):
    n, f = zp.shape

    def body(p0_ref, p1_ref, z_ref, d_ref, b_ref, o_ref):
        dinv = _dinv(d_ref)
        h = (p0_ref[...] + p1_ref[...] + z_ref[...]) * dinv + b_ref[...]
        o_ref[...] = jnp.maximum(h, 0.0) * dinv

    return pl.pallas_call(
        body,
        grid=(n // _R,),
        in_specs=[
            pl.BlockSpec((_R, f), lambda i: (i, 0)),
            pl.BlockSpec((_R, f), lambda i: (i, 0)),
            pl.BlockSpec((_R, f), lambda i: (i, 0)),
            pl.BlockSpec((_R, _NC), lambda i: (i, 0)),
            pl.BlockSpec((1, f), lambda i: (0, 0)),
        ],
        out_specs=pl.BlockSpec((_R, f), lambda i: (i, 0)),
        out_shape=jax.ShapeDtypeStruct((n, f), jnp.float32),
    )(p0, p1, zp, deg_t, b)


def _tc_last(p0, p1, hp, deg_t, b, w3):
    n, f = hp.shape
    f_out = w3.shape[1]

    def body(p0_ref, p1_ref, h_ref, d_ref, b_ref, w_ref, o_ref):
        dinv = _dinv(d_ref)
        t = (p0_ref[...] + p1_ref[...] + h_ref[...]) * dinv
        o = jnp.dot(t, w_ref[...],
                    preferred_element_type=jnp.float32) + b_ref[...]
        m = jnp.max(o, axis=1, keepdims=True)
        lse = jnp.log(jnp.sum(jnp.exp(o - m), axis=1, keepdims=True)) + m
        o_ref[...] = o - lse

    return pl.pallas_call(
        body,
        grid=(n // _R,),
        in_specs=[
            pl.BlockSpec((_R, f), lambda i: (i, 0)),
            pl.BlockSpec((_R, f), lambda i: (i, 0)),
            pl.BlockSpec((_R, f), lambda i: (i, 0)),
            pl.BlockSpec((_R, _NC), lambda i: (i, 0)),
            pl.BlockSpec((1, f_out), lambda i: (0, 0)),
            pl.BlockSpec((f, f_out), lambda i: (0, 0)),
        ],
        out_specs=pl.BlockSpec((_R, f_out), lambda i: (i, 0)),
        out_shape=jax.ShapeDtypeStruct((n, f_out), jnp.float32),
    )(p0, p1, hp, deg_t, b, w3)


def kernel(x, edge_index, W1, b1, W2, b2, W3, b3):
    n = x.shape[0]
    ei = edge_index.astype(jnp.int32)
    src, dst = ei[0], ei[1]

    ones_c = jnp.ones((_CH,), jnp.float32)
    zeros_c = jnp.zeros((_CH,), jnp.float32)

    degp = _degree_partials(dst, ones_c, zeros_c, n)
    deg_t = degp.reshape(_NC, n).T

    z1 = _tc_first(x, W1, deg_t)
    p1 = _aggregate(src, dst, z1, jnp.zeros((_CH, z1.shape[1]), jnp.float32), n)
    z2 = _tc_mid(p1[:n], p1[n:], z1, deg_t, b1.reshape(1, -1), W2)
    p2 = _aggregate(src, dst, z2, jnp.zeros((_CH, z2.shape[1]), jnp.float32), n)
    h2p = _tc_prelast(p2[:n], p2[n:], z2, deg_t, b2.reshape(1, -1))
    p3 = _aggregate(src, dst, h2p, jnp.zeros((_CH, h2p.shape[1]), jnp.float32), n)
    out = _tc_last(p3[:n], p3[n:], h2p, deg_t, b3.reshape(1, -1), W3)
    return out


# speedup vs baseline: 37.5703x; 2.0997x over previous
import functools

import jax
import jax.numpy as jnp
from jax import lax
from jax.experimental import pallas as pl
from jax.experimental.pallas import tpu as pltpu
from jax.experimental.pallas import tpu_sc as plsc

_NC = 2
_NS = 16
_NW = _NC * _NS
_CH = 128
_PAD_ROWS = 16


def _sc_mesh():
    return plsc.VectorSubcoreMesh(
        core_axis_name="c", subcore_axis_name="s",
        num_cores=_NC, num_subcores=_NS)


def _degree_partials(dst2d, ones_src, zeros_src, n):
    cw = dst2d.shape[0] // _NW
    n_acc = n + _PAD_ROWS
    nz, ztail = divmod(n, _CH)
    nzc = (nz + _NS - 1) // _NS

    @functools.partial(
        pl.kernel,
        out_type=jax.ShapeDtypeStruct((_NC * n,), jnp.float32),
        mesh=_sc_mesh(),
        scratch_types=[
            pltpu.VMEM((cw, _CH), jnp.int32),
            pltpu.VMEM((_CH,), jnp.float32),
            pltpu.VMEM((_CH,), jnp.float32),
            pltpu.VMEM_SHARED((n_acc,), jnp.float32),
        ],
        compiler_params=pltpu.CompilerParams(use_tc_tiling_on_sc=False),
    )
    def deg_kernel(dst_hbm, ones_hbm, zeros_hbm, out_hbm,
                   didx, ones_v, zeros_v, acc):
        c = lax.axis_index("c")
        s = lax.axis_index("s")
        w = c * _NS + s
        pltpu.sync_copy(ones_hbm, ones_v)
        pltpu.sync_copy(zeros_hbm, zeros_v)
        pltpu.sync_copy(dst_hbm.at[pl.ds(w * cw, cw)], didx)

        def zacc(j, carry):
            i = s + j * _NS

            @pl.when(i < nz)
            def _():
                pltpu.sync_copy(zeros_v, acc.at[pl.ds(i * _CH, _CH)])
            return carry
        lax.fori_loop(0, nzc, zacc, 0)
        if ztail:
            @pl.when(s == 0)
            def _():
                pltpu.sync_copy(zeros_v.at[pl.ds(0, ztail)],
                                acc.at[pl.ds(nz * _CH, ztail)])
        plsc.subcore_barrier()

        def body(i, carry):
            pltpu.sync_copy(ones_v, acc.at[didx.at[i]], add=True)
            return carry
        lax.fori_loop(0, cw, body, 0)
        plsc.subcore_barrier()

        def out_body(j, carry):
            i = s + j * _NS

            @pl.when(i < nz)
            def _():
                pltpu.sync_copy(acc.at[pl.ds(i * _CH, _CH)], zeros_v)
                pltpu.sync_copy(zeros_v,
                                out_hbm.at[pl.ds(c * n + i * _CH, _CH)])
            return carry
        lax.fori_loop(0, nzc, out_body, 0)
        if ztail:
            @pl.when(s == 0)
            def _():
                pltpu.sync_copy(acc.at[pl.ds(nz * _CH, ztail)],
                                zeros_v.at[pl.ds(0, ztail)])
                pltpu.sync_copy(zeros_v.at[pl.ds(0, ztail)],
                                out_hbm.at[pl.ds(c * n + nz * _CH, ztail)])

    return deg_kernel(dst2d, ones_src, zeros_src)


def _aggregate(src2d, dst2d, z, zero_rows, n):
    cw = src2d.shape[0] // _NW
    f = z.shape[1]
    n_acc = n + _PAD_ROWS
    nz, ztail = divmod(n, _CH)
    nzc = (nz + _NS - 1) // _NS
    assert cw % 2 == 0

    @functools.partial(
        pl.kernel,
        out_type=jax.ShapeDtypeStruct((_NC * n, f), jnp.float32),
        mesh=_sc_mesh(),
        scratch_types=[
            pltpu.VMEM((cw, _CH), jnp.int32),
            pltpu.VMEM((cw, _CH), jnp.int32),
            pltpu.VMEM((_CH, f), jnp.float32),
            pltpu.VMEM((_CH, f), jnp.float32),
            pltpu.VMEM((_CH, f), jnp.float32),
            pltpu.VMEM_SHARED((n_acc, f), jnp.float32),
            pltpu.SemaphoreType.DMA,
            pltpu.SemaphoreType.DMA,
        ],
        compiler_params=pltpu.CompilerParams(use_tc_tiling_on_sc=False),
    )
    def agg_kernel(src_hbm, dst_hbm, z_hbm, zrows_hbm, out_hbm,
                   sidx, didx, rows0, rows1, zrows, acc, sem0, sem1):
        c = lax.axis_index("c")
        s = lax.axis_index("s")
        w = c * _NS + s
        pltpu.sync_copy(zrows_hbm, zrows)
        pltpu.sync_copy(src_hbm.at[pl.ds(w * cw, cw)], sidx)
        pltpu.sync_copy(dst_hbm.at[pl.ds(w * cw, cw)], didx)

        def zacc(j, carry):
            i = s + j * _NS

            @pl.when(i < nz)
            def _():
                pltpu.sync_copy(zrows, acc.at[pl.ds(i * _CH, _CH)])
            return carry
        lax.fori_loop(0, nzc, zacc, 0)
        if ztail:
            @pl.when(s == 0)
            def _():
                pltpu.sync_copy(zrows.at[pl.ds(0, ztail)],
                                acc.at[pl.ds(nz * _CH, ztail)])
        plsc.subcore_barrier()

        bufs = ((rows0, sem0), (rows1, sem1))
        pltpu.async_copy(z_hbm.at[sidx.at[0]], rows0, sem0)
        pltpu.async_copy(z_hbm.at[sidx.at[1]], rows1, sem1)

        def body(j, carry):
            for b in range(2):
                i = 2 * j + b
                rows, sem = bufs[b]
                pltpu.make_async_copy(z_hbm.at[pl.ds(0, _CH)], rows,
                                      sem).wait()
                pltpu.sync_copy(rows, acc.at[didx.at[i]], add=True)

                @pl.when(i + 2 < cw)
                def _():
                    pltpu.async_copy(z_hbm.at[sidx.at[i + 2]], rows, sem)
            return carry
        lax.fori_loop(0, cw // 2, body, 0)
        plsc.subcore_barrier()

        def out_body(j, carry):
            i = s + j * _NS

            @pl.when(i < nz)
            def _():
                pltpu.sync_copy(acc.at[pl.ds(i * _CH, _CH)], rows0)
                pltpu.sync_copy(rows0,
                                out_hbm.at[pl.ds(c * n + i * _CH, _CH)])
            return carry
        lax.fori_loop(0, nzc, out_body, 0)
        if ztail:
            @pl.when(s == 0)
            def _():
                pltpu.sync_copy(acc.at[pl.ds(nz * _CH, ztail)],
                                rows0.at[pl.ds(0, ztail)])
                pltpu.sync_copy(rows0.at[pl.ds(0, ztail)],
                                out_hbm.at[pl.ds(c * n + nz * _CH, ztail)])

    return agg_kernel(src2d, dst2d, z, zero_rows)


def _dinv(d_ref):
    return lax.rsqrt(d_ref[:, 0:1] + d_ref[:, 1:2] + 1.0)


_R = 1000


def _tc_first(x, w1, deg_t):
    n, d_in = x.shape
    f = w1.shape[1]

    def body(x_ref, w_ref, d_ref, o_ref):
        z = jnp.dot(x_ref[...], w_ref[...],
                    preferred_element_type=jnp.float32)
        o_ref[...] = z * _dinv(d_ref)

    return pl.pallas_call(
        body,
        grid=(n // _R,),
        in_specs=[
            pl.BlockSpec((_R, d_in), lambda i: (i, 0)),
            pl.BlockSpec((d_in, f), lambda i: (0, 0)),
            pl.BlockSpec((_R, _NC), lambda i: (i, 0)),
        ],
        out_specs=pl.BlockSpec((_R, f), lambda i: (i, 0)),
        out_shape=jax.ShapeDtypeStruct((n, f), jnp.float32),
    )(x, w1, deg_t)


def _tc_mid(p0, p1, zp, deg_t, b, w_next):
    n, f = zp.shape
    f2 = w_next.shape[1]

    def body(p0_ref, p1_ref, z_ref, d_ref, b_ref, w_ref, o_ref):
        dinv = _dinv(d_ref)
        h = (p0_ref[...] + p1_ref[...] + z_ref[...]) * dinv + b_ref[...]
        h = jnp.maximum(h, 0.0)
        o_ref[...] = jnp.dot(h, w_ref[...],
                             preferred_element_type=jnp.float32) * dinv

    return pl.pallas_call(
        body,
        grid=(n // _R,),
        in_specs=[
            pl.BlockSpec((_R, f), lambda i: (i, 0)),
            pl.BlockSpec((_R, f), lambda i: (i, 0)),
            pl.BlockSpec((_R, f), lambda i: (i, 0)),
            pl.BlockSpec((_R, _NC), lambda i: (i, 0)),
            pl.BlockSpec((1, f), lambda i: (0, 0)),
            pl.BlockSpec((f, f2), lambda i: (0, 0)),
        ],
        out_specs=pl.BlockSpec((_R, f2), lambda i: (i, 0)),
        out_shape=jax.ShapeDtypeStruct((n, f2), jnp.float32),
    )(p0, p1, zp, deg_t, b, w_next)


def _tc_prelast(p0, p1, zp, deg_t, b):
    n, f = zp.shape

    def body(p0_ref, p1_ref, z_ref, d_ref, b_ref, o_ref):
        dinv = _dinv(d_ref)
        h = (p0_ref[...] + p1_ref[...] + z_ref[...]) * dinv + b_ref[...]
        o_ref[...] = jnp.maximum(h, 0.0) * dinv

    return pl.pallas_call(
        body,
        grid=(n // _R,),
        in_specs=[
            pl.BlockSpec((_R, f), lambda i: (i, 0)),
            pl.BlockSpec((_R, f), lambda i: (i, 0)),
            pl.BlockSpec((_R, f), lambda i: (i, 0)),
            pl.BlockSpec((_R, _NC), lambda i: (i, 0)),
            pl.BlockSpec((1, f), lambda i: (0, 0)),
        ],
        out_specs=pl.BlockSpec((_R, f), lambda i: (i, 0)),
        out_shape=jax.ShapeDtypeStruct((n, f), jnp.float32),
    )(p0, p1, zp, deg_t, b)


def _tc_last(p0, p1, hp, deg_t, b, w3):
    n, f = hp.shape
    f_out = w3.shape[1]

    def body(p0_ref, p1_ref, h_ref, d_ref, b_ref, w_ref, o_ref):
        dinv = _dinv(d_ref)
        t = (p0_ref[...] + p1_ref[...] + h_ref[...]) * dinv
        o = jnp.dot(t, w_ref[...],
                    preferred_element_type=jnp.float32) + b_ref[...]
        m = jnp.max(o, axis=1, keepdims=True)
        lse = jnp.log(jnp.sum(jnp.exp(o - m), axis=1, keepdims=True)) + m
        o_ref[...] = o - lse

    return pl.pallas_call(
        body,
        grid=(n // _R,),
        in_specs=[
            pl.BlockSpec((_R, f), lambda i: (i, 0)),
            pl.BlockSpec((_R, f), lambda i: (i, 0)),
            pl.BlockSpec((_R, f), lambda i: (i, 0)),
            pl.BlockSpec((_R, _NC), lambda i: (i, 0)),
            pl.BlockSpec((1, f_out), lambda i: (0, 0)),
            pl.BlockSpec((f, f_out), lambda i: (0, 0)),
        ],
        out_specs=pl.BlockSpec((_R, f_out), lambda i: (i, 0)),
        out_shape=jax.ShapeDtypeStruct((n, f_out), jnp.float32),
    )(p0, p1, hp, deg_t, b, w3)


def kernel(x, edge_index, W1, b1, W2, b2, W3, b3):
    n = x.shape[0]
    ei = edge_index.astype(jnp.int32)
    src, dst = ei[0], ei[1]
    e = src.shape[0]

    cw = -(-e // (_NW * _CH))
    cw += cw % 2
    pad = _NW * cw * _CH - e
    if pad:
        ar = jnp.arange(pad, dtype=jnp.int32)
        src = jnp.concatenate([src, ar % _CH])
        dst = jnp.concatenate([dst, n + (ar % _PAD_ROWS)])
    src2d = src.reshape(-1, _CH)
    dst2d = dst.reshape(-1, _CH)

    ones_c = jnp.ones((_CH,), jnp.float32)
    zeros_c = jnp.zeros((_CH,), jnp.float32)

    degp = _degree_partials(dst2d, ones_c, zeros_c, n)
    deg_t = degp.reshape(_NC, n).T

    z1 = _tc_first(x, W1, deg_t)
    p1 = _aggregate(src2d, dst2d, z1, jnp.zeros((_CH, z1.shape[1]), jnp.float32), n)
    z2 = _tc_mid(p1[:n], p1[n:], z1, deg_t, b1.reshape(1, -1), W2)
    p2 = _aggregate(src2d, dst2d, z2, jnp.zeros((_CH, z2.shape[1]), jnp.float32), n)
    h2p = _tc_prelast(p2[:n], p2[n:], z2, deg_t, b2.reshape(1, -1))
    p3 = _aggregate(src2d, dst2d, h2p, jnp.zeros((_CH, h2p.shape[1]), jnp.float32), n)
    out = _tc_last(p3[:n], p3[n:], h2p, deg_t, b3.reshape(1, -1), W3)
    return out


# speedup vs baseline: 53.0448x; 1.4119x over previous
import functools

import jax
import jax.numpy as jnp
from jax import lax
from jax.experimental import pallas as pl
from jax.experimental.pallas import tpu as pltpu
from jax.experimental.pallas import tpu_sc as plsc

_NC = 2
_NS = 16
_NW = _NC * _NS
_CH = 128
_PAD_ROWS = 16
_RING = 8
_PREF = 4


def _sc_mesh():
    return plsc.VectorSubcoreMesh(
        core_axis_name="c", subcore_axis_name="s",
        num_cores=_NC, num_subcores=_NS)


def _degree_partials(dst2d, ones_src, zeros_src, n):
    cw = dst2d.shape[0] // _NW
    n_acc = n + _PAD_ROWS
    nz, ztail = divmod(n, _CH)
    nzc = (nz + _NS - 1) // _NS

    @functools.partial(
        pl.kernel,
        out_type=jax.ShapeDtypeStruct((_NC * n,), jnp.float32),
        mesh=_sc_mesh(),
        scratch_types=[
            pltpu.VMEM((cw, _CH), jnp.int32),
            pltpu.VMEM((_CH,), jnp.float32),
            pltpu.VMEM((_CH,), jnp.float32),
            pltpu.VMEM_SHARED((n_acc,), jnp.float32),
            pltpu.SemaphoreType.DMA,
        ],
        compiler_params=pltpu.CompilerParams(use_tc_tiling_on_sc=False),
    )
    def deg_kernel(dst_hbm, ones_hbm, zeros_hbm, out_hbm,
                   didx, ones_v, zeros_v, acc, sem):
        c = lax.axis_index("c")
        s = lax.axis_index("s")
        w = c * _NS + s
        pltpu.sync_copy(ones_hbm, ones_v)
        pltpu.sync_copy(zeros_hbm, zeros_v)
        pltpu.sync_copy(dst_hbm.at[pl.ds(w * cw, cw)], didx)

        def zacc(j, carry):
            i = s + j * _NS

            @pl.when(i < nz)
            def _():
                pltpu.sync_copy(zeros_v, acc.at[pl.ds(i * _CH, _CH)])
            return carry
        lax.fori_loop(0, nzc, zacc, 0)
        if ztail:
            @pl.when(s == 0)
            def _():
                pltpu.sync_copy(zeros_v.at[pl.ds(0, ztail)],
                                acc.at[pl.ds(nz * _CH, ztail)])
        plsc.subcore_barrier()

        def body(j, carry):
            for b in range(8):
                pltpu.async_copy(ones_v, acc.at[didx.at[j * 8 + b]], sem,
                                 add=True)
            for b in range(8):
                pltpu.make_async_copy(ones_v, acc.at[didx.at[0]], sem).wait()
            return carry
        lax.fori_loop(0, cw // 8, body, 0)
        plsc.subcore_barrier()

        def out_body(j, carry):
            i = s + j * _NS

            @pl.when(i < nz)
            def _():
                pltpu.sync_copy(acc.at[pl.ds(i * _CH, _CH)], zeros_v)
                pltpu.sync_copy(zeros_v,
                                out_hbm.at[pl.ds(c * n + i * _CH, _CH)])
            return carry
        lax.fori_loop(0, nzc, out_body, 0)
        if ztail:
            @pl.when(s == 0)
            def _():
                pltpu.sync_copy(acc.at[pl.ds(nz * _CH, ztail)],
                                zeros_v.at[pl.ds(0, ztail)])
                pltpu.sync_copy(zeros_v.at[pl.ds(0, ztail)],
                                out_hbm.at[pl.ds(c * n + nz * _CH, ztail)])

    return deg_kernel(dst2d, ones_src, zeros_src)


def _aggregate(src2d, dst2d, z, zero_rows, n):
    cw = src2d.shape[0] // _NW
    f = z.shape[1]
    n_acc = n + _PAD_ROWS
    nz, ztail = divmod(n, _CH)
    nzc = (nz + _NS - 1) // _NS
    assert cw % _RING == 0 and cw >= _RING

    @functools.partial(
        pl.kernel,
        out_type=jax.ShapeDtypeStruct((_NC * n, f), jnp.float32),
        mesh=_sc_mesh(),
        scratch_types=[
            pltpu.VMEM((cw, _CH), jnp.int32),
            pltpu.VMEM((cw, _CH), jnp.int32),
            [pltpu.VMEM((_CH, f), jnp.float32)] * _RING,
            pltpu.VMEM((_CH, f), jnp.float32),
            pltpu.VMEM_SHARED((n_acc, f), jnp.float32),
            [pltpu.SemaphoreType.DMA] * _RING,
            [pltpu.SemaphoreType.DMA] * _RING,
        ],
        compiler_params=pltpu.CompilerParams(use_tc_tiling_on_sc=False),
    )
    def agg_kernel(src_hbm, dst_hbm, z_hbm, zrows_hbm, out_hbm,
                   sidx, didx, bufs, zrows, acc, gsems, tsems):
        c = lax.axis_index("c")
        s = lax.axis_index("s")
        w = c * _NS + s
        pltpu.sync_copy(zrows_hbm, zrows)
        pltpu.sync_copy(src_hbm.at[pl.ds(w * cw, cw)], sidx)
        pltpu.sync_copy(dst_hbm.at[pl.ds(w * cw, cw)], didx)

        def zacc(j, carry):
            i = s + j * _NS

            @pl.when(i < nz)
            def _():
                pltpu.sync_copy(zrows, acc.at[pl.ds(i * _CH, _CH)])
            return carry
        lax.fori_loop(0, nzc, zacc, 0)
        if ztail:
            @pl.when(s == 0)
            def _():
                pltpu.sync_copy(zrows.at[pl.ds(0, ztail)],
                                acc.at[pl.ds(nz * _CH, ztail)])
        plsc.subcore_barrier()

        for b in range(_PREF):
            pltpu.async_copy(z_hbm.at[sidx.at[b]], bufs[b], gsems[b])

        def body(j, carry):
            for b in range(_RING):
                i = j * _RING + b
                bg = (b + _PREF) % _RING
                nxt = i + _PREF

                @pl.when((nxt < cw) & (i >= _RING - _PREF))
                def _():
                    pltpu.make_async_copy(bufs[bg], acc.at[didx.at[0]],
                                          tsems[bg]).wait()
                    pltpu.async_copy(z_hbm.at[sidx.at[nxt]], bufs[bg],
                                     gsems[bg])

                @pl.when((nxt < cw) & (i < _RING - _PREF))
                def _():
                    pltpu.async_copy(z_hbm.at[sidx.at[nxt]], bufs[bg],
                                     gsems[bg])

                pltpu.make_async_copy(z_hbm.at[pl.ds(0, _CH)], bufs[b],
                                      gsems[b]).wait()
                pltpu.async_copy(bufs[b], acc.at[didx.at[i]], tsems[b],
                                 add=True)
            return carry
        lax.fori_loop(0, cw // _RING, body, 0)
        for b in range(_RING):
            pltpu.make_async_copy(bufs[b], acc.at[didx.at[0]],
                                  tsems[b]).wait()
        plsc.subcore_barrier()

        def out_body(j, carry):
            i = s + j * _NS

            @pl.when(i < nz)
            def _():
                pltpu.sync_copy(acc.at[pl.ds(i * _CH, _CH)], bufs[0])
                pltpu.sync_copy(bufs[0],
                                out_hbm.at[pl.ds(c * n + i * _CH, _CH)])
            return carry
        lax.fori_loop(0, nzc, out_body, 0)
        if ztail:
            @pl.when(s == 0)
            def _():
                pltpu.sync_copy(acc.at[pl.ds(nz * _CH, ztail)],
                                bufs[0].at[pl.ds(0, ztail)])
                pltpu.sync_copy(bufs[0].at[pl.ds(0, ztail)],
                                out_hbm.at[pl.ds(c * n + nz * _CH, ztail)])

    return agg_kernel(src2d, dst2d, z, zero_rows)


def _dinv(d_ref):
    return lax.rsqrt(d_ref[:, 0:1] + d_ref[:, 1:2] + 1.0)


_R = 2000


def _tc_first(x, w1, deg_t):
    n, d_in = x.shape
    f = w1.shape[1]

    def body(x_ref, w_ref, d_ref, o_ref):
        z = jnp.dot(x_ref[...], w_ref[...],
                    preferred_element_type=jnp.float32)
        o_ref[...] = z * _dinv(d_ref)

    return pl.pallas_call(
        body,
        grid=(n // _R,),
        in_specs=[
            pl.BlockSpec((_R, d_in), lambda i: (i, 0)),
            pl.BlockSpec((d_in, f), lambda i: (0, 0)),
            pl.BlockSpec((_R, _NC), lambda i: (i, 0)),
        ],
        out_specs=pl.BlockSpec((_R, f), lambda i: (i, 0)),
        out_shape=jax.ShapeDtypeStruct((n, f), jnp.float32),
    )(x, w1, deg_t)


def _tc_mid(p, zp, deg_t, b, w_next):
    n, f = zp.shape
    f2 = w_next.shape[1]
    nb = n // _R

    def body(p0_ref, p1_ref, z_ref, d_ref, b_ref, w_ref, o_ref):
        dinv = _dinv(d_ref)
        h = (p0_ref[...] + p1_ref[...] + z_ref[...]) * dinv + b_ref[...]
        h = jnp.maximum(h, 0.0)
        o_ref[...] = jnp.dot(h, w_ref[...],
                             preferred_element_type=jnp.float32) * dinv

    return pl.pallas_call(
        body,
        grid=(n // _R,),
        in_specs=[
            pl.BlockSpec((_R, f), lambda i: (i, 0)),
            pl.BlockSpec((_R, f), lambda i: (i + nb, 0)),
            pl.BlockSpec((_R, f), lambda i: (i, 0)),
            pl.BlockSpec((_R, _NC), lambda i: (i, 0)),
            pl.BlockSpec((1, f), lambda i: (0, 0)),
            pl.BlockSpec((f, f2), lambda i: (0, 0)),
        ],
        out_specs=pl.BlockSpec((_R, f2), lambda i: (i, 0)),
        out_shape=jax.ShapeDtypeStruct((n, f2), jnp.float32),
    )(p, p, zp, deg_t, b, w_next)


def _tc_prelast(p, zp, deg_t, b):
    n, f = zp.shape
    nb = n // _R

    def body(p0_ref, p1_ref, z_ref, d_ref, b_ref, o_ref):
        dinv = _dinv(d_ref)
        h = (p0_ref[...] + p1_ref[...] + z_ref[...]) * dinv + b_ref[...]
        o_ref[...] = jnp.maximum(h, 0.0) * dinv

    return pl.pallas_call(
        body,
        grid=(n // _R,),
        in_specs=[
            pl.BlockSpec((_R, f), lambda i: (i, 0)),
            pl.BlockSpec((_R, f), lambda i: (i + nb, 0)),
            pl.BlockSpec((_R, f), lambda i: (i, 0)),
            pl.BlockSpec((_R, _NC), lambda i: (i, 0)),
            pl.BlockSpec((1, f), lambda i: (0, 0)),
        ],
        out_specs=pl.BlockSpec((_R, f), lambda i: (i, 0)),
        out_shape=jax.ShapeDtypeStruct((n, f), jnp.float32),
    )(p, p, zp, deg_t, b)


def _tc_last(p, hp, deg_t, b, w3):
    n, f = hp.shape
    f_out = w3.shape[1]
    nb = n // _R

    def body(p0_ref, p1_ref, h_ref, d_ref, b_ref, w_ref, o_ref):
        dinv = _dinv(d_ref)
        t = (p0_ref[...] + p1_ref[...] + h_ref[...]) * dinv
        o = jnp.dot(t, w_ref[...],
                    preferred_element_type=jnp.float32) + b_ref[...]
        m = jnp.max(o, axis=1, keepdims=True)
        lse = jnp.log(jnp.sum(jnp.exp(o - m), axis=1, keepdims=True)) + m
        o_ref[...] = o - lse

    return pl.pallas_call(
        body,
        grid=(n // _R,),
        in_specs=[
            pl.BlockSpec((_R, f), lambda i: (i, 0)),
            pl.BlockSpec((_R, f), lambda i: (i + nb, 0)),
            pl.BlockSpec((_R, f), lambda i: (i, 0)),
            pl.BlockSpec((_R, _NC), lambda i: (i, 0)),
            pl.BlockSpec((1, f_out), lambda i: (0, 0)),
            pl.BlockSpec((f, f_out), lambda i: (0, 0)),
        ],
        out_specs=pl.BlockSpec((_R, f_out), lambda i: (i, 0)),
        out_shape=jax.ShapeDtypeStruct((n, f_out), jnp.float32),
    )(p, p, hp, deg_t, b, w3)


def kernel(x, edge_index, W1, b1, W2, b2, W3, b3):
    n = x.shape[0]
    ei = edge_index.astype(jnp.int32)
    src, dst = ei[0], ei[1]
    e = src.shape[0]

    cw = -(-e // (_NW * _CH))
    cw += (-cw) % _RING
    pad = _NW * cw * _CH - e
    if pad:
        ar = jnp.arange(pad, dtype=jnp.int32)
        src = jnp.concatenate([src, ar % _CH])
        dst = jnp.concatenate([dst, n + (ar % _PAD_ROWS)])
    src2d = src.reshape(-1, _CH)
    dst2d = dst.reshape(-1, _CH)

    ones_c = jnp.ones((_CH,), jnp.float32)
    zeros_c = jnp.zeros((_CH,), jnp.float32)

    degp = _degree_partials(dst2d, ones_c, zeros_c, n)
    deg_t = degp.reshape(_NC, n).T

    z1 = _tc_first(x, W1, deg_t)
    p1 = _aggregate(src2d, dst2d, z1, jnp.zeros((_CH, z1.shape[1]), jnp.float32), n)
    z2 = _tc_mid(p1, z1, deg_t, b1.reshape(1, -1), W2)
    p2 = _aggregate(src2d, dst2d, z2, jnp.zeros((_CH, z2.shape[1]), jnp.float32), n)
    h2p = _tc_prelast(p2, z2, deg_t, b2.reshape(1, -1))
    p3 = _aggregate(src2d, dst2d, h2p, jnp.zeros((_CH, h2p.shape[1]), jnp.float32), n)
    out = _tc_last(p3, h2p, deg_t, b3.reshape(1, -1), W3)
    return out


# speedup vs baseline: 55.9290x; 1.0544x over previous
import functools

import jax
import jax.numpy as jnp
from jax import lax
from jax.experimental import pallas as pl
from jax.experimental.pallas import tpu as pltpu
from jax.experimental.pallas import tpu_sc as plsc

_NC = 2
_NS = 16
_NW = _NC * _NS
_CH = 128
_PAD_ROWS = 16
_RING = 8
_PREF = 4


def _sc_mesh():
    return plsc.VectorSubcoreMesh(
        core_axis_name="c", subcore_axis_name="s",
        num_cores=_NC, num_subcores=_NS)


def _worker_range(w, nch):
    cmax = -(-nch // _NW)
    lo = (w * nch) // _NW
    hi = ((w + 1) * nch) // _NW
    base = jnp.minimum(lo, nch - cmax)
    return lo - base, hi - lo, base, cmax


def _degree_partials(ei3, ones_src, zeros_src, n):
    nch = ei3.shape[1]
    cmax = -(-nch // _NW)
    n_acc = n + _PAD_ROWS
    nz, ztail = divmod(n, _CH)
    nzc = (nz + _NS - 1) // _NS

    @functools.partial(
        pl.kernel,
        out_type=jax.ShapeDtypeStruct((_NC * n,), jnp.float32),
        mesh=_sc_mesh(),
        scratch_types=[
            pltpu.VMEM((cmax, _CH), jnp.int32),
            pltpu.VMEM((_CH,), jnp.float32),
            pltpu.VMEM((_CH,), jnp.float32),
            pltpu.VMEM_SHARED((n_acc,), jnp.float32),
            pltpu.SemaphoreType.DMA,
        ],
        compiler_params=pltpu.CompilerParams(use_tc_tiling_on_sc=False),
    )
    def deg_kernel(ei_hbm, ones_hbm, zeros_hbm, out_hbm,
                   didx, ones_v, zeros_v, acc, sem):
        c = lax.axis_index("c")
        s = lax.axis_index("s")
        w = c * _NS + s
        off, cnt, base, _ = _worker_range(w, nch)
        pltpu.sync_copy(ones_hbm, ones_v)
        pltpu.sync_copy(zeros_hbm, zeros_v)
        pltpu.sync_copy(ei_hbm.at[1, pl.ds(base, cmax)], didx)

        def zacc(j, carry):
            i = s + j * _NS

            @pl.when(i < nz)
            def _():
                pltpu.sync_copy(zeros_v, acc.at[pl.ds(i * _CH, _CH)])
            return carry
        lax.fori_loop(0, nzc, zacc, 0)
        if ztail:
            @pl.when(s == 0)
            def _():
                pltpu.sync_copy(zeros_v.at[pl.ds(0, ztail)],
                                acc.at[pl.ds(nz * _CH, ztail)])
        plsc.subcore_barrier()

        def body(j, carry):
            for b in range(8):
                i = j * 8 + b

                @pl.when(i < cnt)
                def _():
                    pltpu.async_copy(ones_v, acc.at[didx.at[off + i]], sem,
                                     add=True)
            for b in range(8):
                i = j * 8 + b

                @pl.when(i < cnt)
                def _():
                    pltpu.make_async_copy(ones_v, acc.at[didx.at[0]],
                                          sem).wait()
            return carry
        lax.fori_loop(0, (cmax + 7) // 8, body, 0)
        plsc.subcore_barrier()

        def out_body(j, carry):
            i = s + j * _NS

            @pl.when(i < nz)
            def _():
                pltpu.sync_copy(acc.at[pl.ds(i * _CH, _CH)], zeros_v)
                pltpu.sync_copy(zeros_v,
                                out_hbm.at[pl.ds(c * n + i * _CH, _CH)])
            return carry
        lax.fori_loop(0, nzc, out_body, 0)
        if ztail:
            @pl.when(s == 0)
            def _():
                pltpu.sync_copy(acc.at[pl.ds(nz * _CH, ztail)],
                                zeros_v.at[pl.ds(0, ztail)])
                pltpu.sync_copy(zeros_v.at[pl.ds(0, ztail)],
                                out_hbm.at[pl.ds(c * n + nz * _CH, ztail)])

    return deg_kernel(ei3, ones_src, zeros_src)


def _aggregate(ei3, z, zero_rows, n):
    nch = ei3.shape[1]
    f = z.shape[1]
    n_acc = n + _PAD_ROWS
    nz, ztail = divmod(n, _CH)
    nzc = (nz + _NS - 1) // _NS
    cmax = -(-nch // _NW)
    assert nch // _NW >= _RING

    @functools.partial(
        pl.kernel,
        out_type=jax.ShapeDtypeStruct((_NC * n, f), jnp.float32),
        mesh=_sc_mesh(),
        scratch_types=[
            pltpu.VMEM((cmax, _CH), jnp.int32),
            pltpu.VMEM((cmax, _CH), jnp.int32),
            [pltpu.VMEM((_CH, f), jnp.float32)] * _RING,
            pltpu.VMEM((_CH, f), jnp.float32),
            pltpu.VMEM_SHARED((n_acc, f), jnp.float32),
            [pltpu.SemaphoreType.DMA] * _RING,
            [pltpu.SemaphoreType.DMA] * _RING,
        ],
        compiler_params=pltpu.CompilerParams(use_tc_tiling_on_sc=False),
    )
    def agg_kernel(ei_hbm, z_hbm, zrows_hbm, out_hbm,
                   sidx, didx, bufs, zrows, acc, gsems, tsems):
        c = lax.axis_index("c")
        s = lax.axis_index("s")
        w = c * _NS + s
        off, cnt, base, _ = _worker_range(w, nch)
        pltpu.sync_copy(zrows_hbm, zrows)
        pltpu.sync_copy(ei_hbm.at[0, pl.ds(base, cmax)], sidx)
        pltpu.sync_copy(ei_hbm.at[1, pl.ds(base, cmax)], didx)

        def zacc(j, carry):
            i = s + j * _NS

            @pl.when(i < nz)
            def _():
                pltpu.sync_copy(zrows, acc.at[pl.ds(i * _CH, _CH)])
            return carry
        lax.fori_loop(0, nzc, zacc, 0)
        if ztail:
            @pl.when(s == 0)
            def _():
                pltpu.sync_copy(zrows.at[pl.ds(0, ztail)],
                                acc.at[pl.ds(nz * _CH, ztail)])
        plsc.subcore_barrier()

        for b in range(_PREF):
            pltpu.async_copy(z_hbm.at[sidx.at[off + b]], bufs[b], gsems[b])

        def body(j, carry):
            for b in range(_RING):
                i = j * _RING + b
                bg = (b + _PREF) % _RING
                nxt = i + _PREF

                @pl.when((nxt < cnt) & (i >= _RING - _PREF))
                def _():
                    pltpu.make_async_copy(bufs[bg], acc.at[didx.at[0]],
                                          tsems[bg]).wait()
                    pltpu.async_copy(z_hbm.at[sidx.at[off + nxt]], bufs[bg],
                                     gsems[bg])

                @pl.when((nxt < cnt) & (i < _RING - _PREF))
                def _():
                    pltpu.async_copy(z_hbm.at[sidx.at[off + nxt]], bufs[bg],
                                     gsems[bg])

                @pl.when(i < cnt)
                def _():
                    pltpu.make_async_copy(z_hbm.at[pl.ds(0, _CH)], bufs[b],
                                          gsems[b]).wait()
                    pltpu.async_copy(bufs[b], acc.at[didx.at[off + i]],
                                     tsems[b], add=True)
            return carry
        lax.fori_loop(0, (cmax + _RING - 1) // _RING, body, 0)
        for b in range(_RING):
            pltpu.make_async_copy(bufs[b], acc.at[didx.at[0]],
                                  tsems[b]).wait()
        plsc.subcore_barrier()

        def out_body(j, carry):
            i = s + j * _NS

            @pl.when(i < nz)
            def _():
                pltpu.sync_copy(acc.at[pl.ds(i * _CH, _CH)], bufs[0])
                pltpu.sync_copy(bufs[0],
                                out_hbm.at[pl.ds(c * n + i * _CH, _CH)])
            return carry
        lax.fori_loop(0, nzc, out_body, 0)
        if ztail:
            @pl.when(s == 0)
            def _():
                pltpu.sync_copy(acc.at[pl.ds(nz * _CH, ztail)],
                                bufs[0].at[pl.ds(0, ztail)])
                pltpu.sync_copy(bufs[0].at[pl.ds(0, ztail)],
                                out_hbm.at[pl.ds(c * n + nz * _CH, ztail)])

    return agg_kernel(ei3, z, zero_rows)


def _dinv(d_ref):
    return lax.rsqrt(d_ref[:, 0:1] + d_ref[:, 1:2] + 1.0)


_R = 2000


def _tc_first(x, w1, deg_t):
    n, d_in = x.shape
    f = w1.shape[1]

    def body(x_ref, w_ref, d_ref, o_ref):
        z = jnp.dot(x_ref[...], w_ref[...],
                    preferred_element_type=jnp.float32)
        o_ref[...] = z * _dinv(d_ref)

    return pl.pallas_call(
        body,
        grid=(n // _R,),
        in_specs=[
            pl.BlockSpec((_R, d_in), lambda i: (i, 0)),
            pl.BlockSpec((d_in, f), lambda i: (0, 0)),
            pl.BlockSpec((_R, _NC), lambda i: (i, 0)),
        ],
        out_specs=pl.BlockSpec((_R, f), lambda i: (i, 0)),
        out_shape=jax.ShapeDtypeStruct((n, f), jnp.float32),
    )(x, w1, deg_t)


def _tc_mid(p, zp, deg_t, b, w_next):
    n, f = zp.shape
    f2 = w_next.shape[1]
    nb = n // _R

    def body(p0_ref, p1_ref, z_ref, d_ref, b_ref, w_ref, o_ref):
        dinv = _dinv(d_ref)
        h = (p0_ref[...] + p1_ref[...] + z_ref[...]) * dinv + b_ref[...]
        h = jnp.maximum(h, 0.0)
        o_ref[...] = jnp.dot(h, w_ref[...],
                             preferred_element_type=jnp.float32) * dinv

    return pl.pallas_call(
        body,
        grid=(n // _R,),
        in_specs=[
            pl.BlockSpec((_R, f), lambda i: (i, 0)),
            pl.BlockSpec((_R, f), lambda i: (i + nb, 0)),
            pl.BlockSpec((_R, f), lambda i: (i, 0)),
            pl.BlockSpec((_R, _NC), lambda i: (i, 0)),
            pl.BlockSpec((1, f), lambda i: (0, 0)),
            pl.BlockSpec((f, f2), lambda i: (0, 0)),
        ],
        out_specs=pl.BlockSpec((_R, f2), lambda i: (i, 0)),
        out_shape=jax.ShapeDtypeStruct((n, f2), jnp.float32),
    )(p, p, zp, deg_t, b, w_next)


def _tc_prelast(p, zp, deg_t, b):
    n, f = zp.shape
    nb = n // _R

    def body(p0_ref, p1_ref, z_ref, d_ref, b_ref, o_ref):
        dinv = _dinv(d_ref)
        h = (p0_ref[...] + p1_ref[...] + z_ref[...]) * dinv + b_ref[...]
        o_ref[...] = jnp.maximum(h, 0.0) * dinv

    return pl.pallas_call(
        body,
        grid=(n // _R,),
        in_specs=[
            pl.BlockSpec((_R, f), lambda i: (i, 0)),
            pl.BlockSpec((_R, f), lambda i: (i + nb, 0)),
            pl.BlockSpec((_R, f), lambda i: (i, 0)),
            pl.BlockSpec((_R, _NC), lambda i: (i, 0)),
            pl.BlockSpec((1, f), lambda i: (0, 0)),
        ],
        out_specs=pl.BlockSpec((_R, f), lambda i: (i, 0)),
        out_shape=jax.ShapeDtypeStruct((n, f), jnp.float32),
    )(p, p, zp, deg_t, b)


def _tc_last(p, hp, deg_t, b, w3):
    n, f = hp.shape
    f_out = w3.shape[1]
    nb = n // _R

    def body(p0_ref, p1_ref, h_ref, d_ref, b_ref, w_ref, o_ref):
        dinv = _dinv(d_ref)
        t = (p0_ref[...] + p1_ref[...] + h_ref[...]) * dinv
        o = jnp.dot(t, w_ref[...],
                    preferred_element_type=jnp.float32) + b_ref[...]
        m = jnp.max(o, axis=1, keepdims=True)
        lse = jnp.log(jnp.sum(jnp.exp(o - m), axis=1, keepdims=True)) + m
        o_ref[...] = o - lse

    return pl.pallas_call(
        body,
        grid=(n // _R,),
        in_specs=[
            pl.BlockSpec((_R, f), lambda i: (i, 0)),
            pl.BlockSpec((_R, f), lambda i: (i + nb, 0)),
            pl.BlockSpec((_R, f), lambda i: (i, 0)),
            pl.BlockSpec((_R, _NC), lambda i: (i, 0)),
            pl.BlockSpec((1, f_out), lambda i: (0, 0)),
            pl.BlockSpec((f, f_out), lambda i: (0, 0)),
        ],
        out_specs=pl.BlockSpec((_R, f_out), lambda i: (i, 0)),
        out_shape=jax.ShapeDtypeStruct((n, f_out), jnp.float32),
    )(p, p, hp, deg_t, b, w3)


def kernel(x, edge_index, W1, b1, W2, b2, W3, b3):
    n = x.shape[0]
    ei = edge_index.astype(jnp.int32)
    e = ei.shape[1]

    if e % _CH:
        pad = _CH - e % _CH
        ar = jnp.arange(pad, dtype=jnp.int32)
        src = jnp.concatenate([ei[0], ar % _CH])
        dst = jnp.concatenate([ei[1], n + (ar % _PAD_ROWS)])
        ei3 = jnp.stack([src.reshape(-1, _CH), dst.reshape(-1, _CH)])
    else:
        ei3 = ei.reshape(2, -1, _CH)

    ones_c = jnp.ones((_CH,), jnp.float32)
    zeros_c = jnp.zeros((_CH,), jnp.float32)

    degp = _degree_partials(ei3, ones_c, zeros_c, n)
    deg_t = degp.reshape(_NC, n).T

    z1 = _tc_first(x, W1, deg_t)
    p1 = _aggregate(ei3, z1, jnp.zeros((_CH, z1.shape[1]), jnp.float32), n)
    z2 = _tc_mid(p1, z1, deg_t, b1.reshape(1, -1), W2)
    p2 = _aggregate(ei3, z2, jnp.zeros((_CH, z2.shape[1]), jnp.float32), n)
    h2p = _tc_prelast(p2, z2, deg_t, b2.reshape(1, -1))
    p3 = _aggregate(ei3, h2p, jnp.zeros((_CH, h2p.shape[1]), jnp.float32), n)
    out = _tc_last(p3, h2p, deg_t, b3.reshape(1, -1), W3)
    return out
